# Initial kernel scaffold; baseline (speedup 1.0000x reference)
import jax
import jax.numpy as jnp
from jax.experimental import pallas as pl


def kernel(x, gate_W, gate_b, W1, b1, W2, b2, gate_center):
    raise NotImplementedError("write your pallas kernel here")


# speedup vs baseline: 1.6215x; 1.6215x over previous
import functools

import jax
import jax.numpy as jnp
from jax import lax
from jax.experimental import pallas as pl
from jax.experimental.pallas import tpu as pltpu
from jax.experimental.pallas import tpu_sc as plsc

B, N, C, E, H = 2, 2048, 768, 8, 768
T = B * N
TILE = 128
NBLK = T // TILE
MAXTILES = T // TILE + E
GROWS = MAXTILES * TILE
NWORK = 32
TPW = T // NWORK


def _router_body(x_ref, gw_ref, gb_ref, gc_ref,
                 eid_ref, rank_ref, meta_ref, te_ref, carry_ref):
    i = pl.program_id(0)

    @pl.when(i == 0)
    def _init():
        carry_ref[...] = jnp.zeros_like(carry_ref)

    scores = jnp.dot(x_ref[...], gw_ref[...],
                     preferred_element_type=jnp.float32)
    scores = scores + gb_ref[...] - gc_ref[...]

    lane = lax.broadcasted_iota(jnp.int32, (TILE, E), 1)
    mx = jnp.max(scores, axis=1, keepdims=True)
    e = jnp.min(jnp.where(scores == mx, lane, E), axis=1)

    oh = (e[:, None] == lane).astype(jnp.float32)
    tri = (lax.broadcasted_iota(jnp.int32, (TILE, TILE), 0) >=
           lax.broadcasted_iota(jnp.int32, (TILE, TILE), 1)).astype(jnp.float32)
    incl = jnp.dot(tri, oh, preferred_element_type=jnp.float32)
    carry = carry_ref[...]
    rank = jnp.sum(oh * (incl + carry), axis=1) - 1.0
    new_carry = carry + incl[TILE - 1:TILE, :]
    carry_ref[...] = new_carry

    eid_ref[...] = e.reshape(1, 1, TILE)
    rank_ref[...] = rank.astype(jnp.int32).reshape(1, 1, TILE)

    @pl.when(i == NBLK - 1)
    def _meta():
        counts = new_carry
        ntiles = jnp.floor((counts + (TILE - 1.0)) * (1.0 / TILE))
        le = (lax.broadcasted_iota(jnp.int32, (E, E), 0) <=
              lax.broadcasted_iota(jnp.int32, (E, E), 1)).astype(jnp.float32)
        cum_incl = jnp.dot(ntiles, le, preferred_element_type=jnp.float32)
        cum_excl = cum_incl - ntiles
        seg_start = (cum_excl * float(TILE)).astype(jnp.int32)
        ntot = cum_incl[:, E - 1:E].astype(jnp.int32)
        zeros7 = jnp.zeros((1, 7), jnp.int32)
        meta_ref[...] = jnp.concatenate([seg_start, ntot, zeros7], axis=1)

        jrow = lax.broadcasted_iota(jnp.int32, (TILE, E), 0)
        ge = (jrow >= cum_incl.astype(jnp.int32)).astype(jnp.int32)
        te = jnp.minimum(jnp.sum(ge, axis=1, keepdims=True), E - 1)
        te_ref[...] = te


def _route(x2, gate_W, gb2, gc2):
    return pl.pallas_call(
        _router_body,
        grid=(NBLK,),
        in_specs=[
            pl.BlockSpec((TILE, C), lambda i: (i, 0)),
            pl.BlockSpec((C, E), lambda i: (0, 0)),
            pl.BlockSpec((1, E), lambda i: (0, 0)),
            pl.BlockSpec((1, E), lambda i: (0, 0)),
        ],
        out_specs=[
            pl.BlockSpec((1, 1, TILE), lambda i: (i, 0, 0)),
            pl.BlockSpec((1, 1, TILE), lambda i: (i, 0, 0)),
            pl.BlockSpec((1, 16), lambda i: (0, 0)),
            pl.BlockSpec((TILE, 1), lambda i: (0, 0)),
        ],
        out_shape=[
            jax.ShapeDtypeStruct((NBLK, 1, TILE), jnp.int32),
            jax.ShapeDtypeStruct((NBLK, 1, TILE), jnp.int32),
            jax.ShapeDtypeStruct((1, 16), jnp.int32),
            jax.ShapeDtypeStruct((TILE, 1), jnp.int32),
        ],
        scratch_shapes=[pltpu.VMEM((1, E), jnp.float32)],
    )(x2, gate_W, gb2, gc2)


@functools.cache
def _sc_mesh():
    return plsc.VectorSubcoreMesh(
        core_axis_name="c", subcore_axis_name="s",
        num_cores=2, num_subcores=16)


@functools.cache
def _make_dispatch():
    @functools.partial(
        pl.kernel,
        out_type=(jax.ShapeDtypeStruct((GROWS, C), jnp.float32),
                  jax.ShapeDtypeStruct((T,), jnp.int32)),
        mesh=_sc_mesh(),
        scratch_types=[
            pltpu.VMEM((TPW,), jnp.int32),
            pltpu.VMEM((TPW,), jnp.int32),
            pltpu.VMEM((16,), jnp.int32),
            pltpu.VMEM((TPW,), jnp.int32),
            pltpu.VMEM((TPW, C), jnp.float32),
            pltpu.SemaphoreType.DMA,
        ],
        compiler_params=pltpu.CompilerParams(needs_layout_passes=False),
    )
    def _dispatch(x_hbm, e_hbm, r_hbm, ps_hbm, xg_hbm, dest_hbm,
                  e_v, r_v, ps_v, dest_v, rows_v, sem):
        wid = lax.axis_index("s") * 2 + lax.axis_index("c")
        base = wid * TPW
        pltpu.sync_copy(e_hbm.at[pl.ds(base, TPW)], e_v)
        pltpu.sync_copy(r_hbm.at[pl.ds(base, TPW)], r_v)
        pltpu.sync_copy(ps_hbm, ps_v)
        for i in range(TPW // 16):
            e16 = e_v[pl.ds(i * 16, 16)]
            r16 = r_v[pl.ds(i * 16, 16)]
            seg16 = plsc.load_gather(ps_v, [e16])
            dest_v[pl.ds(i * 16, 16)] = seg16 + r16
        pltpu.sync_copy(dest_v, dest_hbm.at[pl.ds(base, TPW)])
        pltpu.sync_copy(x_hbm.at[pl.ds(base, TPW)], rows_v)
        pltpu.async_copy(rows_v, xg_hbm.at[dest_v], sem).wait()

    return _dispatch


def _mlp_body(te_ref, nt_ref, xg_ref, w1_ref, b1_ref, w2_ref, b2_ref, yg_ref):
    j = pl.program_id(0)

    @pl.when(j < nt_ref[0])
    def _():
        h = jnp.dot(xg_ref[...], w1_ref[0],
                    preferred_element_type=jnp.float32) + b1_ref[0]
        h = jax.nn.gelu(h)
        yg_ref[...] = jnp.dot(h, w2_ref[0],
                              preferred_element_type=jnp.float32) + b2_ref[0]


def _windex(j, te_ref, nt_ref):
    return te_ref[jnp.minimum(j, nt_ref[0] - 1)]


def _grouped_mlp(te, nt, xg, W1, b1, W2, b2):
    grid_spec = pltpu.PrefetchScalarGridSpec(
        num_scalar_prefetch=2,
        grid=(MAXTILES,),
        in_specs=[
            pl.BlockSpec((TILE, C), lambda j, te, nt: (j, 0)),
            pl.BlockSpec((1, C, H), lambda j, te, nt: (_windex(j, te, nt), 0, 0)),
            pl.BlockSpec((1, 1, H), lambda j, te, nt: (_windex(j, te, nt), 0, 0)),
            pl.BlockSpec((1, H, C), lambda j, te, nt: (_windex(j, te, nt), 0, 0)),
            pl.BlockSpec((1, 1, C), lambda j, te, nt: (_windex(j, te, nt), 0, 0)),
        ],
        out_specs=pl.BlockSpec((TILE, C), lambda j, te, nt: (j, 0)),
    )
    return pl.pallas_call(
        _mlp_body,
        grid_spec=grid_spec,
        out_shape=jax.ShapeDtypeStruct((GROWS, C), jnp.float32),
    )(te, nt, xg, W1, b1, W2, b2)


@functools.cache
def _make_combine():
    @functools.partial(
        pl.kernel,
        out_type=jax.ShapeDtypeStruct((T, C), jnp.float32),
        mesh=_sc_mesh(),
        scratch_types=[
            pltpu.VMEM((TPW,), jnp.int32),
            pltpu.VMEM((TPW, C), jnp.float32),
            pltpu.SemaphoreType.DMA,
        ],
    )
    def _combine(yg_hbm, dest_hbm, out_hbm, dest_v, rows_v, sem):
        wid = lax.axis_index("s") * 2 + lax.axis_index("c")
        base = wid * TPW
        pltpu.sync_copy(dest_hbm.at[pl.ds(base, TPW)], dest_v)
        pltpu.async_copy(yg_hbm.at[dest_v], rows_v, sem).wait()
        pltpu.sync_copy(rows_v, out_hbm.at[pl.ds(base, TPW)])

    return _combine


def kernel(x, gate_W, gate_b, W1, b1, W2, b2, gate_center):
    x2 = x.reshape(T, C)
    gb2 = gate_b.reshape(1, E)
    gc2 = gate_center.reshape(1, E)

    eids, ranks, meta, te = _route(x2, gate_W, gb2, gc2)
    eflat = eids.reshape(T)
    rflat = ranks.reshape(T)
    ps16 = meta.reshape(16)
    nt = meta.reshape(16)[E:E + 1]
    te40 = te.reshape(TILE)[:MAXTILES]

    xg, dest = _make_dispatch()(x2, eflat, rflat, ps16)
    yg = _grouped_mlp(te40, nt, xg, W1,
                      b1.reshape(E, 1, H), W2, b2.reshape(E, 1, C))
    out = _make_combine()(yg, dest)
    return out.reshape(B, N, C)


# speedup vs baseline: 2.0568x; 1.2685x over previous
import functools

import numpy as _np

import jax
import jax.numpy as jnp
from jax import lax
from jax.experimental import pallas as pl
from jax.experimental.pallas import tpu as pltpu
from jax.experimental.pallas import tpu_sc as plsc

B, N, C, E, H = 2, 2048, 768, 8, 768
T = B * N
TILE = 256
RBLK = 512
NBLK = T // RBLK
MAXTILES = T // TILE + E
GROWS = MAXTILES * TILE
NWORK = 32
TPW = T // NWORK
TEM = 128


def _router_body(x_ref, gw_ref, gb_ref, gc_ref, tri_ref,
                 eid_ref, rank_ref, meta_ref, te_ref, carry_ref):
    i = pl.program_id(0)

    @pl.when(i == 0)
    def _init():
        carry_ref[...] = jnp.zeros_like(carry_ref)

    scores = lax.dot_general(gw_ref[...], x_ref[...],
                             (((0,), (1,)), ((), ())),
                             preferred_element_type=jnp.float32)
    scores = scores + gb_ref[...] - gc_ref[...]

    row = lax.broadcasted_iota(jnp.int32, (E, RBLK), 0)
    mx = jnp.max(scores, axis=0, keepdims=True)
    e = jnp.min(jnp.where(scores == mx, row, E), axis=0,
                keepdims=True)

    oh = (e == row).astype(jnp.float32)
    incl = jnp.dot(oh, tri_ref[...], preferred_element_type=jnp.float32)
    carry = carry_ref[...]
    rank = jnp.sum(oh * (incl + carry), axis=0,
                   keepdims=True) - 1.0
    new_carry = carry + incl[:, RBLK - 1:RBLK]
    carry_ref[...] = new_carry

    eid_ref[...] = e.reshape(1, 1, RBLK)
    rank_ref[...] = rank.astype(jnp.int32).reshape(1, 1, RBLK)

    @pl.when(i == NBLK - 1)
    def _meta():
        counts = new_carry
        ntiles = jnp.floor((counts + (TILE - 1.0)) * (1.0 / TILE))
        ge8 = (lax.broadcasted_iota(jnp.int32, (E, E), 0) >=
               lax.broadcasted_iota(jnp.int32, (E, E), 1)).astype(jnp.float32)
        cum_incl = jnp.dot(ge8, ntiles, preferred_element_type=jnp.float32)
        cum_excl = cum_incl - ntiles
        eye = (lax.broadcasted_iota(jnp.int32, (E, E), 0) ==
               lax.broadcasted_iota(jnp.int32, (E, E), 1)).astype(jnp.float32)
        seg_row = jnp.dot(jnp.ones((1, E), jnp.float32), eye * cum_excl,
                          preferred_element_type=jnp.float32) * float(TILE)
        ntot = cum_incl[E - 1:E, :].astype(jnp.int32)
        zeros7 = jnp.zeros((1, 7), jnp.int32)
        meta_ref[...] = jnp.concatenate(
            [seg_row.astype(jnp.int32), ntot, zeros7], axis=1)

        jcol = lax.broadcasted_iota(jnp.int32, (E, TEM), 1)
        ge = (jcol >= cum_incl.astype(jnp.int32)).astype(jnp.int32)
        te = jnp.minimum(jnp.sum(ge, axis=0, keepdims=True), E - 1)
        te_ref[...] = te
        te_ref[:, TEM - 1:TEM] = ntot


def _route(x2, gate_W, gb2, gc2, tri):
    return pl.pallas_call(
        _router_body,
        grid=(NBLK,),
        in_specs=[
            pl.BlockSpec((RBLK, C), lambda i: (i, 0)),
            pl.BlockSpec((C, E), lambda i: (0, 0)),
            pl.BlockSpec((E, 1), lambda i: (0, 0)),
            pl.BlockSpec((E, 1), lambda i: (0, 0)),
            pl.BlockSpec((RBLK, RBLK), lambda i: (0, 0)),
        ],
        out_specs=[
            pl.BlockSpec((1, 1, RBLK), lambda i: (i, 0, 0)),
            pl.BlockSpec((1, 1, RBLK), lambda i: (i, 0, 0)),
            pl.BlockSpec((1, 16), lambda i: (0, 0)),
            pl.BlockSpec((1, TEM), lambda i: (0, 0)),
        ],
        out_shape=[
            jax.ShapeDtypeStruct((NBLK, 1, RBLK), jnp.int32),
            jax.ShapeDtypeStruct((NBLK, 1, RBLK), jnp.int32),
            jax.ShapeDtypeStruct((1, 16), jnp.int32),
            jax.ShapeDtypeStruct((1, TEM), jnp.int32),
        ],
        scratch_shapes=[pltpu.VMEM((E, 1), jnp.float32)],
    )(x2, gate_W, gb2, gc2, tri)


@functools.cache
def _sc_mesh():
    return plsc.VectorSubcoreMesh(
        core_axis_name="c", subcore_axis_name="s",
        num_cores=2, num_subcores=16)


@functools.cache
def _make_dispatch():
    @functools.partial(
        pl.kernel,
        out_type=(jax.ShapeDtypeStruct((GROWS, C), jnp.float32),
                  jax.ShapeDtypeStruct((T,), jnp.int32)),
        mesh=_sc_mesh(),
        scratch_types=[
            pltpu.VMEM((TPW,), jnp.int32),
            pltpu.VMEM((TPW,), jnp.int32),
            pltpu.VMEM((16,), jnp.int32),
            pltpu.VMEM((TPW,), jnp.int32),
            pltpu.VMEM((TPW, C), jnp.float32),
            pltpu.SemaphoreType.DMA,
        ],
        compiler_params=pltpu.CompilerParams(needs_layout_passes=False),
    )
    def _dispatch(x_hbm, e_hbm, r_hbm, ps_hbm, xg_hbm, dest_hbm,
                  e_v, r_v, ps_v, dest_v, rows_v, sem):
        wid = lax.axis_index("s") * 2 + lax.axis_index("c")
        base = wid * TPW
        rows_dma = pltpu.async_copy(x_hbm.at[pl.ds(base, TPW)], rows_v, sem)
        pltpu.sync_copy(e_hbm.at[pl.ds(base, TPW)], e_v)
        pltpu.sync_copy(r_hbm.at[pl.ds(base, TPW)], r_v)
        pltpu.sync_copy(ps_hbm, ps_v)
        for i in range(TPW // 16):
            e16 = e_v[pl.ds(i * 16, 16)]
            r16 = r_v[pl.ds(i * 16, 16)]
            seg16 = plsc.load_gather(ps_v, [e16])
            dest_v[pl.ds(i * 16, 16)] = seg16 + r16
        pltpu.sync_copy(dest_v, dest_hbm.at[pl.ds(base, TPW)])
        rows_dma.wait()
        pltpu.async_copy(rows_v, xg_hbm.at[dest_v], sem).wait()

    return _dispatch


def _mlp_body(te_ref, xg_ref, w1_ref, b1_ref, w2_ref, b2_ref, yg_ref):
    j = pl.program_id(0)

    @pl.when(j < te_ref[TEM - 1])
    def _():
        h = jnp.dot(xg_ref[...], w1_ref[0],
                    preferred_element_type=jnp.float32) + b1_ref[0]
        h = jax.nn.gelu(h)
        yg_ref[...] = jnp.dot(h, w2_ref[0],
                              preferred_element_type=jnp.float32) + b2_ref[0]


def _grouped_mlp(te, xg, W1, b1, W2, b2):
    grid_spec = pltpu.PrefetchScalarGridSpec(
        num_scalar_prefetch=1,
        grid=(MAXTILES,),
        in_specs=[
            pl.BlockSpec((TILE, C), lambda j, te: (j, 0)),
            pl.BlockSpec((1, C, H), lambda j, te: (te[j], 0, 0)),
            pl.BlockSpec((1, 1, H), lambda j, te: (te[j], 0, 0)),
            pl.BlockSpec((1, H, C), lambda j, te: (te[j], 0, 0)),
            pl.BlockSpec((1, 1, C), lambda j, te: (te[j], 0, 0)),
        ],
        out_specs=pl.BlockSpec((TILE, C), lambda j, te: (j, 0)),
    )
    return pl.pallas_call(
        _mlp_body,
        grid_spec=grid_spec,
        out_shape=jax.ShapeDtypeStruct((GROWS, C), jnp.float32),
    )(te, xg, W1, b1, W2, b2)


@functools.cache
def _make_combine():
    @functools.partial(
        pl.kernel,
        out_type=jax.ShapeDtypeStruct((T, C), jnp.float32),
        mesh=_sc_mesh(),
        scratch_types=[
            pltpu.VMEM((TPW,), jnp.int32),
            pltpu.VMEM((TPW, C), jnp.float32),
            pltpu.SemaphoreType.DMA,
        ],
    )
    def _combine(yg_hbm, dest_hbm, out_hbm, dest_v, rows_v, sem):
        wid = lax.axis_index("s") * 2 + lax.axis_index("c")
        base = wid * TPW
        pltpu.sync_copy(dest_hbm.at[pl.ds(base, TPW)], dest_v)
        pltpu.async_copy(yg_hbm.at[dest_v], rows_v, sem).wait()
        pltpu.sync_copy(rows_v, out_hbm.at[pl.ds(base, TPW)])

    return _combine


def kernel(x, gate_W, gate_b, W1, b1, W2, b2, gate_center):
    x2 = x.reshape(T, C)
    gb2 = gate_b.reshape(E, 1)
    gc2 = gate_center.reshape(E, 1)

    tri = jnp.asarray(
        _np.triu(_np.ones((RBLK, RBLK), _np.float32)))

    eids, ranks, meta, te = _route(x2, gate_W, gb2, gc2, tri)
    eflat = eids.reshape(T)
    rflat = ranks.reshape(T)
    ps16 = meta.reshape(16)

    xg, dest = _make_dispatch()(x2, eflat, rflat, ps16)
    yg = _grouped_mlp(te.reshape(TEM), xg,
                      W1, b1.reshape(E, 1, H), W2, b2.reshape(E, 1, C))
    out = _make_combine()(yg, dest)
    return out.reshape(B, N, C)


# speedup vs baseline: 2.0822x; 1.0123x over previous
import functools

import numpy as _np

import jax
import jax.numpy as jnp
from jax import lax
from jax.experimental import pallas as pl
from jax.experimental.pallas import tpu as pltpu
from jax.experimental.pallas import tpu_sc as plsc

B, N, C, E, H = 2, 2048, 768, 8, 768
T = B * N
TILE = 256
RBLK = 1024
NBLK = T // RBLK
MAXTILES = T // TILE + E
GROWS = MAXTILES * TILE
NWORK = 32
TPW = T // NWORK
TEM = 128


def _router_body(x_ref, gw_ref, gb_ref, gc_ref, tri_ref,
                 eid_ref, rank_ref, meta_ref, te_ref, carry_ref):
    i = pl.program_id(0)

    @pl.when(i == 0)
    def _init():
        carry_ref[...] = jnp.zeros_like(carry_ref)

    scores = lax.dot_general(gw_ref[...], x_ref[...],
                             (((0,), (1,)), ((), ())),
                             preferred_element_type=jnp.float32)
    scores = scores + gb_ref[...] - gc_ref[...]

    row = lax.broadcasted_iota(jnp.int32, (E, RBLK), 0)
    mx = jnp.max(scores, axis=0, keepdims=True)
    e = jnp.min(jnp.where(scores == mx, row, E), axis=0,
                keepdims=True)

    oh = (e == row).astype(jnp.float32)
    incl = jnp.dot(oh, tri_ref[...], preferred_element_type=jnp.float32)
    carry = carry_ref[...]
    rank = jnp.sum(oh * (incl + carry), axis=0,
                   keepdims=True) - 1.0
    new_carry = carry + incl[:, RBLK - 1:RBLK]
    carry_ref[...] = new_carry

    eid_ref[...] = e.reshape(1, 1, RBLK)
    rank_ref[...] = rank.astype(jnp.int32).reshape(1, 1, RBLK)

    @pl.when(i == NBLK - 1)
    def _meta():
        counts = new_carry
        ntiles = jnp.floor((counts + (TILE - 1.0)) * (1.0 / TILE))
        ge8 = (lax.broadcasted_iota(jnp.int32, (E, E), 0) >=
               lax.broadcasted_iota(jnp.int32, (E, E), 1)).astype(jnp.float32)
        cum_incl = jnp.dot(ge8, ntiles, preferred_element_type=jnp.float32)
        cum_excl = cum_incl - ntiles
        eye = (lax.broadcasted_iota(jnp.int32, (E, E), 0) ==
               lax.broadcasted_iota(jnp.int32, (E, E), 1)).astype(jnp.float32)
        seg_row = jnp.dot(jnp.ones((1, E), jnp.float32), eye * cum_excl,
                          preferred_element_type=jnp.float32) * float(TILE)
        ntot = cum_incl[E - 1:E, :].astype(jnp.int32)
        zeros7 = jnp.zeros((1, 7), jnp.int32)
        meta_ref[...] = jnp.concatenate(
            [seg_row.astype(jnp.int32), ntot, zeros7], axis=1)

        jcol = lax.broadcasted_iota(jnp.int32, (E, TEM), 1)
        ge = (jcol >= cum_incl.astype(jnp.int32)).astype(jnp.int32)
        te = jnp.minimum(jnp.sum(ge, axis=0, keepdims=True), E - 1)
        te_ref[...] = te
        te_ref[:, TEM - 1:TEM] = ntot


def _route(x2, gate_W, gb2, gc2, tri):
    return pl.pallas_call(
        _router_body,
        grid=(NBLK,),
        in_specs=[
            pl.BlockSpec((RBLK, C), lambda i: (i, 0)),
            pl.BlockSpec((C, E), lambda i: (0, 0)),
            pl.BlockSpec((E, 1), lambda i: (0, 0)),
            pl.BlockSpec((E, 1), lambda i: (0, 0)),
            pl.BlockSpec((RBLK, RBLK), lambda i: (0, 0)),
        ],
        out_specs=[
            pl.BlockSpec((1, 1, RBLK), lambda i: (i, 0, 0)),
            pl.BlockSpec((1, 1, RBLK), lambda i: (i, 0, 0)),
            pl.BlockSpec((1, 16), lambda i: (0, 0)),
            pl.BlockSpec((1, TEM), lambda i: (0, 0)),
        ],
        out_shape=[
            jax.ShapeDtypeStruct((NBLK, 1, RBLK), jnp.int32),
            jax.ShapeDtypeStruct((NBLK, 1, RBLK), jnp.int32),
            jax.ShapeDtypeStruct((1, 16), jnp.int32),
            jax.ShapeDtypeStruct((1, TEM), jnp.int32),
        ],
        scratch_shapes=[pltpu.VMEM((E, 1), jnp.float32)],
    )(x2, gate_W, gb2, gc2, tri)


@functools.cache
def _sc_mesh():
    return plsc.VectorSubcoreMesh(
        core_axis_name="c", subcore_axis_name="s",
        num_cores=2, num_subcores=16)


@functools.cache
def _make_dispatch():
    @functools.partial(
        pl.kernel,
        out_type=(jax.ShapeDtypeStruct((GROWS, C), jnp.float32),
                  jax.ShapeDtypeStruct((T,), jnp.int32)),
        mesh=_sc_mesh(),
        scratch_types=[
            pltpu.VMEM((TPW,), jnp.int32),
            pltpu.VMEM((TPW,), jnp.int32),
            pltpu.VMEM((16,), jnp.int32),
            pltpu.VMEM((TPW,), jnp.int32),
            pltpu.VMEM((TPW, C), jnp.float32),
            pltpu.SemaphoreType.DMA,
        ],
        compiler_params=pltpu.CompilerParams(needs_layout_passes=False),
    )
    def _dispatch(x_hbm, e_hbm, r_hbm, ps_hbm, xg_hbm, dest_hbm,
                  e_v, r_v, ps_v, dest_v, rows_v, sem):
        wid = lax.axis_index("s") * 2 + lax.axis_index("c")
        base = wid * TPW
        rows_dma = pltpu.async_copy(x_hbm.at[pl.ds(base, TPW)], rows_v, sem)
        pltpu.sync_copy(e_hbm.at[pl.ds(base, TPW)], e_v)
        pltpu.sync_copy(r_hbm.at[pl.ds(base, TPW)], r_v)
        pltpu.sync_copy(ps_hbm, ps_v)
        for i in range(TPW // 16):
            e16 = e_v[pl.ds(i * 16, 16)]
            r16 = r_v[pl.ds(i * 16, 16)]
            seg16 = plsc.load_gather(ps_v, [e16])
            dest_v[pl.ds(i * 16, 16)] = seg16 + r16
        pltpu.sync_copy(dest_v, dest_hbm.at[pl.ds(base, TPW)])
        rows_dma.wait()
        pltpu.async_copy(rows_v, xg_hbm.at[dest_v], sem).wait()

    return _dispatch


def _mlp_body(te_ref, xg_ref, w1_ref, b1_ref, w2_ref, b2_ref, yg_ref):
    j = pl.program_id(0)

    @pl.when(j < te_ref[TEM - 1])
    def _():
        h = jnp.dot(xg_ref[...], w1_ref[0],
                    preferred_element_type=jnp.float32) + b1_ref[0]
        h = jax.nn.gelu(h)
        yg_ref[...] = jnp.dot(h, w2_ref[0],
                              preferred_element_type=jnp.float32) + b2_ref[0]


def _grouped_mlp(te, xg, W1, b1, W2, b2):
    grid_spec = pltpu.PrefetchScalarGridSpec(
        num_scalar_prefetch=1,
        grid=(MAXTILES,),
        in_specs=[
            pl.BlockSpec((TILE, C), lambda j, te: (j, 0)),
            pl.BlockSpec((1, C, H), lambda j, te: (te[j], 0, 0)),
            pl.BlockSpec((1, 1, H), lambda j, te: (te[j], 0, 0)),
            pl.BlockSpec((1, H, C), lambda j, te: (te[j], 0, 0)),
            pl.BlockSpec((1, 1, C), lambda j, te: (te[j], 0, 0)),
        ],
        out_specs=pl.BlockSpec((TILE, C), lambda j, te: (j, 0)),
    )
    return pl.pallas_call(
        _mlp_body,
        grid_spec=grid_spec,
        out_shape=jax.ShapeDtypeStruct((GROWS, C), jnp.float32),
    )(te, xg, W1, b1, W2, b2)


@functools.cache
def _make_combine():
    @functools.partial(
        pl.kernel,
        out_type=jax.ShapeDtypeStruct((T, C), jnp.float32),
        mesh=_sc_mesh(),
        scratch_types=[
            pltpu.VMEM((TPW,), jnp.int32),
            pltpu.VMEM((TPW, C), jnp.float32),
            pltpu.SemaphoreType.DMA,
        ],
    )
    def _combine(yg_hbm, dest_hbm, out_hbm, dest_v, rows_v, sem):
        wid = lax.axis_index("s") * 2 + lax.axis_index("c")
        base = wid * TPW
        pltpu.sync_copy(dest_hbm.at[pl.ds(base, TPW)], dest_v)
        pltpu.async_copy(yg_hbm.at[dest_v], rows_v, sem).wait()
        pltpu.sync_copy(rows_v, out_hbm.at[pl.ds(base, TPW)])

    return _combine


def kernel(x, gate_W, gate_b, W1, b1, W2, b2, gate_center):
    x2 = x.reshape(T, C)
    gb2 = gate_b.reshape(E, 1)
    gc2 = gate_center.reshape(E, 1)

    tri = jnp.asarray(
        _np.triu(_np.ones((RBLK, RBLK), _np.float32)))

    eids, ranks, meta, te = _route(x2, gate_W, gb2, gc2, tri)
    eflat = eids.reshape(T)
    rflat = ranks.reshape(T)
    ps16 = meta.reshape(16)

    xg, dest = _make_dispatch()(x2, eflat, rflat, ps16)
    yg = _grouped_mlp(te.reshape(TEM), xg,
                      W1, b1.reshape(E, 1, H), W2, b2.reshape(E, 1, C))
    out = _make_combine()(yg, dest)
    return out.reshape(B, N, C)


# speedup vs baseline: 2.2046x; 1.0588x over previous
import functools

import numpy as _np

import jax
import jax.numpy as jnp
from jax import lax
from jax.experimental import pallas as pl
from jax.experimental.pallas import tpu as pltpu
from jax.experimental.pallas import tpu_sc as plsc

B, N, C, E, H = 2, 2048, 768, 8, 768
T = B * N
TILE = 512
RBLK = 1024
NBLK = T // RBLK
MAXTILES = T // TILE + E
GROWS = MAXTILES * TILE
NWORK = 32
TPW = T // NWORK
TEM = 128


def _router_body(x_ref, gw_ref, gb_ref, gc_ref, tri_ref,
                 eid_ref, rank_ref, meta_ref, te_ref, carry_ref):
    i = pl.program_id(0)

    @pl.when(i == 0)
    def _init():
        carry_ref[...] = jnp.zeros_like(carry_ref)

    scores = lax.dot_general(gw_ref[...], x_ref[...],
                             (((0,), (1,)), ((), ())),
                             preferred_element_type=jnp.float32)
    scores = scores + gb_ref[...] - gc_ref[...]

    row = lax.broadcasted_iota(jnp.int32, (E, RBLK), 0)
    mx = jnp.max(scores, axis=0, keepdims=True)
    e = jnp.min(jnp.where(scores == mx, row, E), axis=0,
                keepdims=True)

    oh = (e == row).astype(jnp.float32)
    incl = jnp.dot(oh, tri_ref[...], preferred_element_type=jnp.float32)
    carry = carry_ref[...]
    rank = jnp.sum(oh * (incl + carry), axis=0,
                   keepdims=True) - 1.0
    new_carry = carry + incl[:, RBLK - 1:RBLK]
    carry_ref[...] = new_carry

    eid_ref[...] = e.reshape(1, 1, RBLK)
    rank_ref[...] = rank.astype(jnp.int32).reshape(1, 1, RBLK)

    @pl.when(i == NBLK - 1)
    def _meta():
        counts = new_carry
        ntiles = jnp.floor((counts + (TILE - 1.0)) * (1.0 / TILE))
        ge8 = (lax.broadcasted_iota(jnp.int32, (E, E), 0) >=
               lax.broadcasted_iota(jnp.int32, (E, E), 1)).astype(jnp.float32)
        cum_incl = jnp.dot(ge8, ntiles, preferred_element_type=jnp.float32)
        cum_excl = cum_incl - ntiles
        eye = (lax.broadcasted_iota(jnp.int32, (E, E), 0) ==
               lax.broadcasted_iota(jnp.int32, (E, E), 1)).astype(jnp.float32)
        seg_row = jnp.dot(jnp.ones((1, E), jnp.float32), eye * cum_excl,
                          preferred_element_type=jnp.float32) * float(TILE)
        ntot = cum_incl[E - 1:E, :].astype(jnp.int32)
        zeros7 = jnp.zeros((1, 7), jnp.int32)
        meta_ref[...] = jnp.concatenate(
            [seg_row.astype(jnp.int32), ntot, zeros7], axis=1)

        jcol = lax.broadcasted_iota(jnp.int32, (E, TEM), 1)
        ge = (jcol >= cum_incl.astype(jnp.int32)).astype(jnp.int32)
        te = jnp.minimum(jnp.sum(ge, axis=0, keepdims=True), E - 1)
        te_ref[...] = te
        te_ref[:, TEM - 1:TEM] = ntot


def _route(x2, gate_W, gb2, gc2, tri):
    return pl.pallas_call(
        _router_body,
        grid=(NBLK,),
        in_specs=[
            pl.BlockSpec((RBLK, C), lambda i: (i, 0)),
            pl.BlockSpec((C, E), lambda i: (0, 0)),
            pl.BlockSpec((E, 1), lambda i: (0, 0)),
            pl.BlockSpec((E, 1), lambda i: (0, 0)),
            pl.BlockSpec((RBLK, RBLK), lambda i: (0, 0)),
        ],
        out_specs=[
            pl.BlockSpec((1, 1, RBLK), lambda i: (i, 0, 0)),
            pl.BlockSpec((1, 1, RBLK), lambda i: (i, 0, 0)),
            pl.BlockSpec((1, 16), lambda i: (0, 0)),
            pl.BlockSpec((1, TEM), lambda i: (0, 0)),
        ],
        out_shape=[
            jax.ShapeDtypeStruct((NBLK, 1, RBLK), jnp.int32),
            jax.ShapeDtypeStruct((NBLK, 1, RBLK), jnp.int32),
            jax.ShapeDtypeStruct((1, 16), jnp.int32),
            jax.ShapeDtypeStruct((1, TEM), jnp.int32),
        ],
        scratch_shapes=[pltpu.VMEM((E, 1), jnp.float32)],
    )(x2, gate_W, gb2, gc2, tri)


@functools.cache
def _sc_mesh():
    return plsc.VectorSubcoreMesh(
        core_axis_name="c", subcore_axis_name="s",
        num_cores=2, num_subcores=16)


@functools.cache
def _make_dispatch():
    @functools.partial(
        pl.kernel,
        out_type=(jax.ShapeDtypeStruct((GROWS, C), jnp.float32),
                  jax.ShapeDtypeStruct((T,), jnp.int32)),
        mesh=_sc_mesh(),
        scratch_types=[
            pltpu.VMEM((TPW,), jnp.int32),
            pltpu.VMEM((TPW,), jnp.int32),
            pltpu.VMEM((16,), jnp.int32),
            pltpu.VMEM((TPW,), jnp.int32),
            pltpu.VMEM((TPW, C), jnp.float32),
            pltpu.SemaphoreType.DMA,
        ],
        compiler_params=pltpu.CompilerParams(needs_layout_passes=False),
    )
    def _dispatch(x_hbm, e_hbm, r_hbm, ps_hbm, xg_hbm, dest_hbm,
                  e_v, r_v, ps_v, dest_v, rows_v, sem):
        wid = lax.axis_index("s") * 2 + lax.axis_index("c")
        base = wid * TPW
        rows_dma = pltpu.async_copy(x_hbm.at[pl.ds(base, TPW)], rows_v, sem)
        pltpu.sync_copy(e_hbm.at[pl.ds(base, TPW)], e_v)
        pltpu.sync_copy(r_hbm.at[pl.ds(base, TPW)], r_v)
        pltpu.sync_copy(ps_hbm, ps_v)
        for i in range(TPW // 16):
            e16 = e_v[pl.ds(i * 16, 16)]
            r16 = r_v[pl.ds(i * 16, 16)]
            seg16 = plsc.load_gather(ps_v, [e16])
            dest_v[pl.ds(i * 16, 16)] = seg16 + r16
        pltpu.sync_copy(dest_v, dest_hbm.at[pl.ds(base, TPW)])
        rows_dma.wait()
        pltpu.async_copy(rows_v, xg_hbm.at[dest_v], sem).wait()

    return _dispatch


def _mlp_body(te_ref, xg_ref, w1_ref, b1_ref, w2_ref, b2_ref, yg_ref):
    j = pl.program_id(0)

    @pl.when(j < te_ref[TEM - 1])
    def _():
        h = jnp.dot(xg_ref[...], w1_ref[0],
                    preferred_element_type=jnp.float32) + b1_ref[0]
        h = jax.nn.gelu(h)
        yg_ref[...] = jnp.dot(h, w2_ref[0],
                              preferred_element_type=jnp.float32) + b2_ref[0]


def _grouped_mlp(te, xg, W1, b1, W2, b2):
    grid_spec = pltpu.PrefetchScalarGridSpec(
        num_scalar_prefetch=1,
        grid=(MAXTILES,),
        in_specs=[
            pl.BlockSpec((TILE, C), lambda j, te: (j, 0)),
            pl.BlockSpec((1, C, H), lambda j, te: (te[j], 0, 0)),
            pl.BlockSpec((1, 1, H), lambda j, te: (te[j], 0, 0)),
            pl.BlockSpec((1, H, C), lambda j, te: (te[j], 0, 0)),
            pl.BlockSpec((1, 1, C), lambda j, te: (te[j], 0, 0)),
        ],
        out_specs=pl.BlockSpec((TILE, C), lambda j, te: (j, 0)),
    )
    return pl.pallas_call(
        _mlp_body,
        grid_spec=grid_spec,
        out_shape=jax.ShapeDtypeStruct((GROWS, C), jnp.float32),
    )(te, xg, W1, b1, W2, b2)


@functools.cache
def _make_combine():
    @functools.partial(
        pl.kernel,
        out_type=jax.ShapeDtypeStruct((T, C), jnp.float32),
        mesh=_sc_mesh(),
        scratch_types=[
            pltpu.VMEM((TPW,), jnp.int32),
            pltpu.VMEM((TPW, C), jnp.float32),
            pltpu.SemaphoreType.DMA,
        ],
    )
    def _combine(yg_hbm, dest_hbm, out_hbm, dest_v, rows_v, sem):
        wid = lax.axis_index("s") * 2 + lax.axis_index("c")
        base = wid * TPW
        pltpu.sync_copy(dest_hbm.at[pl.ds(base, TPW)], dest_v)
        pltpu.async_copy(yg_hbm.at[dest_v], rows_v, sem).wait()
        pltpu.sync_copy(rows_v, out_hbm.at[pl.ds(base, TPW)])

    return _combine


def kernel(x, gate_W, gate_b, W1, b1, W2, b2, gate_center):
    x2 = x.reshape(T, C)
    gb2 = gate_b.reshape(E, 1)
    gc2 = gate_center.reshape(E, 1)

    tri = jnp.asarray(
        _np.triu(_np.ones((RBLK, RBLK), _np.float32)))

    eids, ranks, meta, te = _route(x2, gate_W, gb2, gc2, tri)
    eflat = eids.reshape(T)
    rflat = ranks.reshape(T)
    ps16 = meta.reshape(16)

    xg, dest = _make_dispatch()(x2, eflat, rflat, ps16)
    yg = _grouped_mlp(te.reshape(TEM), xg,
                      W1, b1.reshape(E, 1, H), W2, b2.reshape(E, 1, C))
    out = _make_combine()(yg, dest)
    return out.reshape(B, N, C)
